# baseline (speedup 1.0000x reference)
import jax
import jax.numpy as jnp
from jax.experimental import pallas as pl
from jax.experimental.pallas import tpu as pltpu


def _gcn_stack_kernel(rpa_hbm, x_ref, Wi_ref, W0_ref, W1_ref, Wo_ref,
                      bi_ref, b0_ref, b1_ref, bo_ref, out_ref,
                      raw0, raw1, sem0, sem1):
    n = raw0.shape[0]
    c0 = pltpu.make_async_copy(rpa_hbm.at[0], raw0, sem0)
    c1 = pltpu.make_async_copy(rpa_hbm.at[1], raw1, sem1)
    c0.start()
    c1.start()

    row = jax.lax.broadcasted_iota(jnp.int32, (n, n), 0)
    col = jax.lax.broadcasted_iota(jnp.int32, (n, n), 1)
    offdiag = row != col
    ones = jnp.ones((1, n), jnp.bfloat16)

    def graph(raw_ref, x, o_ref):
        a = jnp.where((raw_ref[...] != 0) & offdiag,
                      1.0, 0.0).astype(jnp.bfloat16)
        deg = jnp.dot(ones, a, preferred_element_type=jnp.float32) + 1.0
        dis = jax.lax.rsqrt(deg)

        def layer(xt, w_ref, b_ref):
            ht = jax.lax.dot_general(
                w_ref[...].astype(jnp.bfloat16), xt.astype(jnp.bfloat16),
                (((0,), (0,)), ((), ())), preferred_element_type=jnp.float32)
            y = dis * ht
            z = jnp.dot(y.astype(jnp.bfloat16), a,
                        preferred_element_type=jnp.float32) + y
            return jnp.maximum(dis * z + jnp.transpose(b_ref[...]), 0.0)

        xt = jnp.transpose(x)
        r1 = layer(xt, Wi_ref, bi_ref)
        r2 = layer(r1, W0_ref, b0_ref)
        r3 = layer(r2, W1_ref, b1_ref)
        r4 = layer(r3, Wo_ref, bo_ref)
        o_ref[...] = jnp.transpose((r1 + r2 + r3 + r4) * 0.25)

    c0.wait()
    graph(raw0, x_ref[0], out_ref.at[0])
    c1.wait()
    graph(raw1, x_ref[1], out_ref.at[1])


def kernel(aa_rep, rpa, W_in, b_in, W_h0, b_h0, W_h1, b_h1, W_out, b_out):
    B, N, Fi = aa_rep.shape
    Fo = W_out.shape[1]
    ws = [W_in, W_h0, W_h1, W_out]
    brows = [b.reshape(1, -1) for b in (b_in, b_h0, b_h1, b_out)]

    return pl.pallas_call(
        _gcn_stack_kernel,
        in_specs=[
            pl.BlockSpec(memory_space=pltpu.MemorySpace.HBM),
            pl.BlockSpec(memory_space=pltpu.MemorySpace.VMEM),
            *[pl.BlockSpec(memory_space=pltpu.MemorySpace.VMEM)
              for _ in range(8)],
        ],
        out_specs=pl.BlockSpec(memory_space=pltpu.MemorySpace.VMEM),
        out_shape=jax.ShapeDtypeStruct((B, N, Fo), jnp.float32),
        scratch_shapes=[
            pltpu.VMEM((N, N), jnp.int32),
            pltpu.VMEM((N, N), jnp.int32),
            pltpu.SemaphoreType.DMA,
            pltpu.SemaphoreType.DMA,
        ],
    )(rpa, aa_rep, *ws, *brows)


# speedup vs baseline: 1.0251x; 1.0251x over previous
import jax
import jax.numpy as jnp
from jax.experimental import pallas as pl
from jax.experimental.pallas import tpu as pltpu

_CHUNKS = 8


def _gcn_stack_kernel(rpa_hbm, x_ref, Wi_ref, W0_ref, W1_ref, Wo_ref,
                      bi_ref, b0_ref, b1_ref, bo_ref, out_hbm,
                      raw0, raw1, ob0, ob1, a_ref,
                      sem0, sem1, osem0, osem1):
    n = raw0.shape[0]
    ck = n // _CHUNKS
    chunk_copies = [
        pltpu.make_async_copy(rpa_hbm.at[0, pl.ds(j * ck, ck), :],
                              raw0.at[pl.ds(j * ck, ck), :], sem0.at[j])
        for j in range(_CHUNKS)
    ]
    for c in chunk_copies:
        c.start()
    c1 = pltpu.make_async_copy(rpa_hbm.at[1], raw1, sem1)

    ones = jnp.ones((1, n), jnp.bfloat16)

    def mask_chunk(raw_ref, j):
        rows = raw_ref[pl.ds(j * ck, ck), :]
        row = jax.lax.broadcasted_iota(jnp.int32, (ck, n), 0) + j * ck
        col = jax.lax.broadcasted_iota(jnp.int32, (ck, n), 1)
        a_ref[pl.ds(j * ck, ck), :] = jnp.where(
            (rows != 0) & (row != col), 1.0, 0.0).astype(jnp.bfloat16)

    def layers(x):
        a = a_ref[...]
        deg = jnp.dot(ones, a, preferred_element_type=jnp.float32) + 1.0
        dis = jax.lax.rsqrt(deg)

        def layer(xt, w_ref, b_ref):
            ht = jax.lax.dot_general(
                w_ref[...].astype(jnp.bfloat16), xt.astype(jnp.bfloat16),
                (((0,), (0,)), ((), ())), preferred_element_type=jnp.float32)
            y = dis * ht
            z = jnp.dot(y.astype(jnp.bfloat16), a,
                        preferred_element_type=jnp.float32) + y
            return jnp.maximum(dis * z + jnp.transpose(b_ref[...]), 0.0)

        xt = jnp.transpose(x)
        r1 = layer(xt, Wi_ref, bi_ref)
        r2 = layer(r1, W0_ref, b0_ref)
        r3 = layer(r2, W1_ref, b1_ref)
        r4 = layer(r3, Wo_ref, bo_ref)
        return jnp.transpose((r1 + r2 + r3 + r4) * 0.25)

    for j, c in enumerate(chunk_copies):
        c.wait()
        mask_chunk(raw0, j)
    c1.start()
    ob0[...] = layers(x_ref[0])
    st0 = pltpu.make_async_copy(ob0, out_hbm.at[0], osem0)
    st0.start()

    c1.wait()
    for j in range(_CHUNKS):
        mask_chunk(raw1, j)
    ob1[...] = layers(x_ref[1])
    st1 = pltpu.make_async_copy(ob1, out_hbm.at[1], osem1)
    st1.start()
    st0.wait()
    st1.wait()


def kernel(aa_rep, rpa, W_in, b_in, W_h0, b_h0, W_h1, b_h1, W_out, b_out):
    B, N, Fi = aa_rep.shape
    Fo = W_out.shape[1]
    ws = [W_in, W_h0, W_h1, W_out]
    brows = [b.reshape(1, -1) for b in (b_in, b_h0, b_h1, b_out)]

    return pl.pallas_call(
        _gcn_stack_kernel,
        in_specs=[
            pl.BlockSpec(memory_space=pltpu.MemorySpace.HBM),
            pl.BlockSpec(memory_space=pltpu.MemorySpace.VMEM),
            *[pl.BlockSpec(memory_space=pltpu.MemorySpace.VMEM)
              for _ in range(8)],
        ],
        out_specs=pl.BlockSpec(memory_space=pltpu.MemorySpace.HBM),
        out_shape=jax.ShapeDtypeStruct((B, N, Fo), jnp.float32),
        scratch_shapes=[
            pltpu.VMEM((N, N), jnp.int32),
            pltpu.VMEM((N, N), jnp.int32),
            pltpu.VMEM((N, Fo), jnp.float32),
            pltpu.VMEM((N, Fo), jnp.float32),
            pltpu.VMEM((N, N), jnp.bfloat16),
            pltpu.SemaphoreType.DMA((_CHUNKS,)),
            pltpu.SemaphoreType.DMA,
            pltpu.SemaphoreType.DMA,
            pltpu.SemaphoreType.DMA,
        ],
    )(rpa, aa_rep, *ws, *brows)


# speedup vs baseline: 1.0737x; 1.0473x over previous
import jax
import jax.numpy as jnp
from jax.experimental import pallas as pl
from jax.experimental.pallas import tpu as pltpu


def _gcn_stack_kernel(rpa_hbm, x_ref, Wi_ref, W0_ref, W1_ref, Wo_ref,
                      bi_ref, b0_ref, b1_ref, bo_ref, out_hbm,
                      raw0, raw1, ob0, ob1,
                      sem0, sem1, osem0, osem1):
    n = raw0.shape[0]
    c0 = pltpu.make_async_copy(rpa_hbm.at[0], raw0, sem0)
    c1 = pltpu.make_async_copy(rpa_hbm.at[1], raw1, sem1)
    c0.start()

    row = jax.lax.broadcasted_iota(jnp.int32, (n, n), 0)
    col = jax.lax.broadcasted_iota(jnp.int32, (n, n), 1)
    offd = jnp.where(row != col, 1.0, 0.0).astype(jnp.bfloat16)
    ones = jnp.ones((1, n), jnp.bfloat16)

    def wmat(xt, w_ref):
        return jax.lax.dot_general(
            w_ref[...].astype(jnp.bfloat16), xt.astype(jnp.bfloat16),
            (((0,), (0,)), ((), ())), preferred_element_type=jnp.float32)

    xt0 = jnp.transpose(x_ref[0])
    xt1 = jnp.transpose(x_ref[1])
    h1_0 = wmat(xt0, Wi_ref)
    h1_1 = wmat(xt1, Wi_ref)

    def graph(raw_ref, h1):
        a = raw_ref[...].astype(jnp.bfloat16) * offd
        deg = jnp.dot(ones, a, preferred_element_type=jnp.float32) + 1.0
        dis = jax.lax.rsqrt(deg)

        def agg(ht, b_ref):
            y = dis * ht
            z = jnp.dot(y.astype(jnp.bfloat16), a,
                        preferred_element_type=jnp.float32) + y
            return jnp.maximum(dis * z + jnp.transpose(b_ref[...]), 0.0)

        r1 = agg(h1, bi_ref)
        r2 = agg(wmat(r1, W0_ref), b0_ref)
        r3 = agg(wmat(r2, W1_ref), b1_ref)
        r4 = agg(wmat(r3, Wo_ref), bo_ref)
        return jnp.transpose((r1 + r2 + r3 + r4) * 0.25)

    c0.wait()
    c1.start()
    ob0[...] = graph(raw0, h1_0)
    st0 = pltpu.make_async_copy(ob0, out_hbm.at[0], osem0)
    st0.start()

    c1.wait()
    ob1[...] = graph(raw1, h1_1)
    st1 = pltpu.make_async_copy(ob1, out_hbm.at[1], osem1)
    st1.start()
    st0.wait()
    st1.wait()


def kernel(aa_rep, rpa, W_in, b_in, W_h0, b_h0, W_h1, b_h1, W_out, b_out):
    B, N, Fi = aa_rep.shape
    Fo = W_out.shape[1]
    ws = [W_in, W_h0, W_h1, W_out]
    brows = [b.reshape(1, -1) for b in (b_in, b_h0, b_h1, b_out)]

    return pl.pallas_call(
        _gcn_stack_kernel,
        in_specs=[
            pl.BlockSpec(memory_space=pltpu.MemorySpace.HBM),
            pl.BlockSpec(memory_space=pltpu.MemorySpace.VMEM),
            *[pl.BlockSpec(memory_space=pltpu.MemorySpace.VMEM)
              for _ in range(8)],
        ],
        out_specs=pl.BlockSpec(memory_space=pltpu.MemorySpace.HBM),
        out_shape=jax.ShapeDtypeStruct((B, N, Fo), jnp.float32),
        scratch_shapes=[
            pltpu.VMEM((N, N), jnp.int32),
            pltpu.VMEM((N, N), jnp.int32),
            pltpu.VMEM((N, Fo), jnp.float32),
            pltpu.VMEM((N, Fo), jnp.float32),
            pltpu.SemaphoreType.DMA,
            pltpu.SemaphoreType.DMA,
            pltpu.SemaphoreType.DMA,
            pltpu.SemaphoreType.DMA,
        ],
    )(rpa, aa_rep, *ws, *brows)


# speedup vs baseline: 1.0848x; 1.0104x over previous
import jax
import jax.numpy as jnp
from jax.experimental import pallas as pl
from jax.experimental.pallas import tpu as pltpu


def _gcn_stack_kernel(rpa_hbm, x_ref, Wi_ref, W0_ref, W1_ref, Wo_ref,
                      bi_ref, b0_ref, b1_ref, bo_ref, out_hbm,
                      raw0, raw1, ob0, ob1,
                      sem0, sem1, osem0, osem1):
    n = raw0.shape[0]
    c0 = pltpu.make_async_copy(rpa_hbm.at[0], raw0, sem0)
    c1 = pltpu.make_async_copy(rpa_hbm.at[1], raw1, sem1)
    c0.start()

    row = jax.lax.broadcasted_iota(jnp.int32, (n, n), 0)
    col = jax.lax.broadcasted_iota(jnp.int32, (n, n), 1)
    eye = jnp.where(row == col, 1.0, 0.0).astype(jnp.bfloat16)
    offd = jnp.bfloat16(1.0) - eye
    ones = jnp.ones((1, n), jnp.bfloat16)
    bts = [jnp.transpose(b[...]) for b in (bi_ref, b0_ref, b1_ref, bo_ref)]

    def wmat(xt, w_ref):
        return jax.lax.dot_general(
            w_ref[...].astype(jnp.bfloat16), xt.astype(jnp.bfloat16),
            (((0,), (0,)), ((), ())), preferred_element_type=jnp.float32)

    xt0 = jnp.transpose(x_ref[0])
    xt1 = jnp.transpose(x_ref[1])
    h1_0 = wmat(xt0, Wi_ref)
    h1_1 = wmat(xt1, Wi_ref)

    def graph(raw_ref, h1):
        a = raw_ref[...].astype(jnp.bfloat16) * offd + eye
        deg = jnp.dot(ones, a, preferred_element_type=jnp.float32)
        dis = jax.lax.rsqrt(deg)

        def agg(ht, bt):
            yb = (dis * ht).astype(jnp.bfloat16)
            z = jnp.dot(yb, a, preferred_element_type=jnp.float32)
            return jnp.maximum(dis * z + bt, 0.0)

        r1 = agg(h1, bts[0])
        r2 = agg(wmat(r1, W0_ref), bts[1])
        r3 = agg(wmat(r2, W1_ref), bts[2])
        r4 = agg(wmat(r3, Wo_ref), bts[3])
        return jnp.transpose((r1 + r2 + r3 + r4) * 0.25)

    c0.wait()
    c1.start()
    ob0[...] = graph(raw0, h1_0)
    st0 = pltpu.make_async_copy(ob0, out_hbm.at[0], osem0)
    st0.start()

    c1.wait()
    ob1[...] = graph(raw1, h1_1)
    st1 = pltpu.make_async_copy(ob1, out_hbm.at[1], osem1)
    st1.start()
    st0.wait()
    st1.wait()


def kernel(aa_rep, rpa, W_in, b_in, W_h0, b_h0, W_h1, b_h1, W_out, b_out):
    B, N, Fi = aa_rep.shape
    Fo = W_out.shape[1]
    ws = [W_in, W_h0, W_h1, W_out]
    brows = [b.reshape(1, -1) for b in (b_in, b_h0, b_h1, b_out)]

    return pl.pallas_call(
        _gcn_stack_kernel,
        in_specs=[
            pl.BlockSpec(memory_space=pltpu.MemorySpace.HBM),
            pl.BlockSpec(memory_space=pltpu.MemorySpace.VMEM),
            *[pl.BlockSpec(memory_space=pltpu.MemorySpace.VMEM)
              for _ in range(8)],
        ],
        out_specs=pl.BlockSpec(memory_space=pltpu.MemorySpace.HBM),
        out_shape=jax.ShapeDtypeStruct((B, N, Fo), jnp.float32),
        scratch_shapes=[
            pltpu.VMEM((N, N), jnp.int32),
            pltpu.VMEM((N, N), jnp.int32),
            pltpu.VMEM((N, Fo), jnp.float32),
            pltpu.VMEM((N, Fo), jnp.float32),
            pltpu.SemaphoreType.DMA,
            pltpu.SemaphoreType.DMA,
            pltpu.SemaphoreType.DMA,
            pltpu.SemaphoreType.DMA,
        ],
    )(rpa, aa_rep, *ws, *brows)


# speedup vs baseline: 1.1498x; 1.0599x over previous
import jax
import jax.numpy as jnp
from jax.experimental import pallas as pl


def _gcn_stack_kernel(rpa_ref, x_ref, Wi_ref, W0_ref, W1_ref, Wo_ref,
                      bi_ref, b0_ref, b1_ref, bo_ref, out_ref):
    n = rpa_ref.shape[1]
    row = jax.lax.broadcasted_iota(jnp.int32, (n, n), 0)
    col = jax.lax.broadcasted_iota(jnp.int32, (n, n), 1)
    eye = jnp.where(row == col, 1.0, 0.0).astype(jnp.bfloat16)
    offd = jnp.bfloat16(1.0) - eye
    ones = jnp.ones((1, n), jnp.bfloat16)
    bts = [jnp.transpose(b[...]) for b in (bi_ref, b0_ref, b1_ref, bo_ref)]

    a = rpa_ref[0].astype(jnp.bfloat16) * offd + eye
    deg = jnp.dot(ones, a, preferred_element_type=jnp.float32)
    dis = jax.lax.rsqrt(deg)

    def wmat(xt, w_ref):
        return jax.lax.dot_general(
            w_ref[...].astype(jnp.bfloat16), xt.astype(jnp.bfloat16),
            (((0,), (0,)), ((), ())), preferred_element_type=jnp.float32)

    def agg(ht, bt):
        yb = (dis * ht).astype(jnp.bfloat16)
        z = jnp.dot(yb, a, preferred_element_type=jnp.float32)
        return jnp.maximum(dis * z + bt, 0.0)

    xt = jnp.transpose(x_ref[0])
    r1 = agg(wmat(xt, Wi_ref), bts[0])
    r2 = agg(wmat(r1, W0_ref), bts[1])
    r3 = agg(wmat(r2, W1_ref), bts[2])
    r4 = agg(wmat(r3, Wo_ref), bts[3])
    out_ref[0] = jnp.transpose((r1 + r2 + r3 + r4) * 0.25)


def kernel(aa_rep, rpa, W_in, b_in, W_h0, b_h0, W_h1, b_h1, W_out, b_out):
    B, N, Fi = aa_rep.shape
    Fo = W_out.shape[1]
    ws = [W_in, W_h0, W_h1, W_out]
    brows = [b.reshape(1, -1) for b in (b_in, b_h0, b_h1, b_out)]

    def rep_spec(shape):
        return pl.BlockSpec(shape, lambda i: (0,) * len(shape))

    return pl.pallas_call(
        _gcn_stack_kernel,
        grid=(B,),
        in_specs=[
            pl.BlockSpec((1, N, N), lambda i: (i, 0, 0)),
            pl.BlockSpec((1, N, Fi), lambda i: (i, 0, 0)),
            *[rep_spec(w.shape) for w in ws],
            *[rep_spec(b.shape) for b in brows],
        ],
        out_specs=pl.BlockSpec((1, N, Fo), lambda i: (i, 0, 0)),
        out_shape=jax.ShapeDtypeStruct((B, N, Fo), jnp.float32),
    )(rpa, aa_rep, *ws, *brows)
